# Initial kernel scaffold; baseline (speedup 1.0000x reference)
import functools

import jax
import jax.numpy as jnp
from jax import lax
from jax.experimental import pallas as pl
from jax.experimental.pallas import tpu as pltpu
from jax.experimental.pallas import tpu_sc as plsc

B, T, D, S = 16, 2048, 768, 8
V = 100
CHUNK = 64
HALF = T // 2
NCHUNK = HALF // CHUNK
ROWS_PER_B = S + 1
ACC_ROWS = B * ROWS_PER_B
DUMP_ROW = ACC_ROWS
VIS_PER_CORE = V // 2


def _make_sc_kernel():
  mesh = plsc.VectorSubcoreMesh(core_axis_name="c", subcore_axis_name="s")

  @functools.partial(
      pl.kernel,
      out_type=jax.ShapeDtypeStruct((2, ACC_ROWS, D), jnp.float32),
      mesh=mesh,
      scratch_types=[
          pltpu.VMEM((CHUNK, D), jnp.float32),
          pltpu.VMEM((CHUNK, D), jnp.float32),
          pltpu.VMEM((CHUNK,), jnp.int32),
          pltpu.VMEM((CHUNK,), jnp.int32),
          pltpu.VMEM((CHUNK,), jnp.int32),
          pltpu.VMEM((CHUNK,), jnp.int32),
          pltpu.VMEM((CHUNK,), jnp.int32),
          pltpu.VMEM_SHARED((ACC_ROWS + 16, D), jnp.float32),
          pltpu.SemaphoreType.DMA,
          pltpu.SemaphoreType.DMA,
          pltpu.SemaphoreType.DMA,
          pltpu.SemaphoreType.DMA,
      ],
  )
  def sc_kernel(feat_hbm, mask_hbm, zeros_hbm, out_hbm,
                data0, data1, mbuf0, mbuf1, ibuf0, ibuf1, vidx, acc,
                semd0, semd1, semm0, semm1):
    c = lax.axis_index("c")
    s = lax.axis_index("s")
    data = (data0, data1)
    mbuf = (mbuf0, mbuf1)
    ibuf = (ibuf0, ibuf1)
    semd = (semd0, semd1)
    semm = (semm0, semm1)

    base = s * ROWS_PER_B
    tok0 = c * HALF

    pltpu.sync_copy(zeros_hbm, acc.at[pl.ds(base, ROWS_PER_B)])

    vis_row = base + S
    for i in range(CHUNK // 16):
      gl = lax.iota(jnp.int32, (16,)) + (i * 16)
      vals = jnp.where(gl < VIS_PER_CORE, vis_row, DUMP_ROW)
      vidx[pl.ds(i * 16, 16)] = vals

    def fill(j, b):
      if j < NCHUNK:
        t = tok0 + j * CHUNK
        d = pltpu.async_copy(feat_hbm.at[s, pl.ds(V + t, CHUNK), :],
                             data[b], semd[b])
        m = pltpu.async_copy(mask_hbm.at[s, pl.ds(t, CHUNK)], mbuf[b], semm[b])
        return (d, m)
      else:
        d = pltpu.async_copy(
            feat_hbm.at[s, pl.ds(c * VIS_PER_CORE, VIS_PER_CORE), :],
            data[b].at[pl.ds(0, VIS_PER_CORE), :], semd[b])
        return (d,)

    NTOT = NCHUNK + 1
    pend = fill(0, 0)
    for j in range(NTOT):
      b = j % 2
      for dsc in pend:
        dsc.wait()
      if j + 1 < NTOT:
        pend = fill(j + 1, 1 - b)
      if j < NCHUNK:
        for i in range(CHUNK // 16):
          mv = mbuf[b][pl.ds(i * 16, 16)]
          rows = mv + (base - 1)
          rows = jnp.where(mv == 0, DUMP_ROW, rows)
          ibuf[b][pl.ds(i * 16, 16)] = rows
        pltpu.sync_copy(data[b], acc.at[ibuf[b]], add=True)
      else:
        pltpu.sync_copy(data[b], acc.at[vidx], add=True)

    pltpu.sync_copy(acc.at[pl.ds(base, ROWS_PER_B)],
                    out_hbm.at[c, pl.ds(base, ROWS_PER_B), :])

  return sc_kernel


_sc_kernel = _make_sc_kernel()


def _tc_body(part_ref, mask_ref, w1_ref, w2_ref, b_ref, out_ref):
  p = part_ref[0, 0] + part_ref[1, 0]
  m = mask_ref[0]
  cnts = [jnp.sum(jnp.where(m == sg, 1.0, 0.0)).reshape(1, 1)
          for sg in range(1, S + 1)]
  counts = jnp.concatenate(cnts, axis=0)
  means = p[0:S, :] / jnp.maximum(counts, 1.0)
  vision = p[S:S + 1, :] * (1.0 / V)
  acc = jnp.dot(means, w1_ref[...], preferred_element_type=jnp.float32)
  vacc = jnp.dot(vision, w2_ref[...], preferred_element_type=jnp.float32)
  out_ref[0] = acc + vacc + b_ref[...]


def _tc_finish(partials, mask, W, b):
  part4 = partials.reshape(2, B, ROWS_PER_B, D)
  mask3 = mask.astype(jnp.int32).reshape(B, 1, T)
  w1 = W[:D]
  w2 = W[D:]
  b2 = b.reshape(1, D)
  out = pl.pallas_call(
      _tc_body,
      grid=(B,),
      in_specs=[
          pl.BlockSpec((2, 1, ROWS_PER_B, D), lambda s: (0, s, 0, 0)),
          pl.BlockSpec((1, 1, T), lambda s: (s, 0, 0)),
          pl.BlockSpec((D, D), lambda s: (0, 0)),
          pl.BlockSpec((D, D), lambda s: (0, 0)),
          pl.BlockSpec((1, D), lambda s: (0, 0)),
      ],
      out_specs=pl.BlockSpec((1, S, D), lambda s: (s, 0, 0)),
      out_shape=jax.ShapeDtypeStruct((B, S, D), jnp.float32),
  )(part4, mask3, w1, w2, b2)
  return out.reshape(B * S, D)


@jax.jit
def kernel(vision_trace_feat, vision_trace_mask, W, b):
  zeros = jnp.zeros((ROWS_PER_B, D), jnp.float32)
  partials = _sc_kernel(vision_trace_feat, vision_trace_mask.astype(jnp.int32),
                        zeros)
  return _tc_finish(partials, vision_trace_mask, W, b)


# speedup vs baseline: 1.7811x; 1.7811x over previous
import functools

import jax
import jax.numpy as jnp
from jax import lax
from jax.experimental import pallas as pl
from jax.experimental.pallas import tpu as pltpu
from jax.experimental.pallas import tpu_sc as plsc

B, T, D, S = 16, 2048, 768, 8
V = 100
CHUNK = 64
HALF = T // 2
NCHUNK = HALF // CHUNK
ROWS_PER_B = 16
ACC_ROWS = B * ROWS_PER_B
DUMP_ROW = ACC_ROWS
VIS_PER_CORE = V // 2


def _make_sc_kernel():
  mesh = plsc.VectorSubcoreMesh(core_axis_name="c", subcore_axis_name="s")

  @functools.partial(
      pl.kernel,
      out_type=jax.ShapeDtypeStruct((2, ACC_ROWS, D), jnp.float32),
      mesh=mesh,
      scratch_types=[
          pltpu.VMEM((CHUNK, D), jnp.float32),
          pltpu.VMEM((CHUNK, D), jnp.float32),
          pltpu.VMEM((CHUNK,), jnp.int32),
          pltpu.VMEM((CHUNK,), jnp.int32),
          pltpu.VMEM((CHUNK,), jnp.int32),
          pltpu.VMEM((CHUNK,), jnp.int32),
          pltpu.VMEM((CHUNK,), jnp.int32),
          pltpu.VMEM_SHARED((ACC_ROWS + 8, D), jnp.float32),
          pltpu.SemaphoreType.DMA,
          pltpu.SemaphoreType.DMA,
          pltpu.SemaphoreType.DMA,
          pltpu.SemaphoreType.DMA,
      ],
      compiler_params=pltpu.CompilerParams(use_tc_tiling_on_sc=False),
  )
  def sc_kernel(feat_hbm, mask_hbm, zeros_hbm, out_hbm,
                data0, data1, mbuf0, mbuf1, ibuf0, ibuf1, vidx, acc,
                semd0, semd1, semm0, semm1):
    c = lax.axis_index("c")
    s = lax.axis_index("s")
    data = (data0, data1)
    mbuf = (mbuf0, mbuf1)
    ibuf = (ibuf0, ibuf1)
    semd = (semd0, semd1)
    semm = (semm0, semm1)

    base = s * ROWS_PER_B
    tok0 = c * HALF

    pltpu.sync_copy(zeros_hbm, acc.at[pl.ds(base, ROWS_PER_B)])

    vis_row = base + S
    for i in range(CHUNK // 16):
      gl = lax.iota(jnp.int32, 16) + (i * 16)
      vals = jnp.where(gl < VIS_PER_CORE, vis_row, DUMP_ROW)
      vidx[pl.ds(i * 16, 16)] = vals

    def fill(j, b):
      if j < NCHUNK:
        t = tok0 + j * CHUNK
        d = pltpu.async_copy(feat_hbm.at[s, pl.ds(V + t, CHUNK), :],
                             data[b], semd[b])
        m = pltpu.async_copy(mask_hbm.at[s, pl.ds(t, CHUNK)], mbuf[b], semm[b])
        return (d, m)
      else:
        d = pltpu.async_copy(
            feat_hbm.at[s, pl.ds(c * VIS_PER_CORE, VIS_PER_CORE), :],
            data[b].at[pl.ds(0, VIS_PER_CORE), :], semd[b])
        return (d,)

    NTOT = NCHUNK + 1
    pend = fill(0, 0)
    for j in range(NTOT):
      b = j % 2
      for dsc in pend:
        dsc.wait()
      if j + 1 < NTOT:
        pend = fill(j + 1, 1 - b)
      if j < NCHUNK:
        for i in range(CHUNK // 16):
          mv = mbuf[b][pl.ds(i * 16, 16)]
          rows = mv + (base - 1)
          rows = jnp.where(mv == 0, DUMP_ROW, rows)
          ibuf[b][pl.ds(i * 16, 16)] = rows
        pltpu.sync_copy(data[b], acc.at[ibuf[b]], add=True)
      else:
        pltpu.sync_copy(data[b], acc.at[vidx], add=True)

    pltpu.sync_copy(acc.at[pl.ds(base, ROWS_PER_B)],
                    out_hbm.at[c, pl.ds(base, ROWS_PER_B), :])

  return sc_kernel


_sc_kernel = _make_sc_kernel()


def _tc_body(part_ref, mask_ref, w1_ref, w2_ref, b_ref, out_ref):
  p = part_ref[0, 0] + part_ref[1, 0]
  m = mask_ref[0]
  cnts = [jnp.sum(jnp.where(m == sg, 1.0, 0.0)).reshape(1, 1)
          for sg in range(1, S + 1)]
  counts = jnp.concatenate(cnts, axis=0)
  means = p[0:S, :] / jnp.maximum(counts, 1.0)
  vision = p[S:S + 1, :] * (1.0 / V)
  acc = jnp.dot(means, w1_ref[...], preferred_element_type=jnp.float32,
                precision=lax.Precision.HIGHEST)
  vacc = jnp.dot(vision, w2_ref[...], preferred_element_type=jnp.float32,
                 precision=lax.Precision.HIGHEST)
  out_ref[0] = acc + vacc + b_ref[...]


def _tc_finish(partials, mask, W, b):
  part4 = partials.reshape(2, B, ROWS_PER_B, D)
  mask3 = mask.astype(jnp.int32).reshape(B, 1, T)
  w1 = W[:D]
  w2 = W[D:]
  b2 = b.reshape(1, D)
  out = pl.pallas_call(
      _tc_body,
      grid=(B,),
      in_specs=[
          pl.BlockSpec((2, 1, ROWS_PER_B, D), lambda s: (0, s, 0, 0)),
          pl.BlockSpec((1, 1, T), lambda s: (s, 0, 0)),
          pl.BlockSpec((D, D), lambda s: (0, 0)),
          pl.BlockSpec((D, D), lambda s: (0, 0)),
          pl.BlockSpec((1, D), lambda s: (0, 0)),
      ],
      out_specs=pl.BlockSpec((1, S, D), lambda s: (s, 0, 0)),
      out_shape=jax.ShapeDtypeStruct((B, S, D), jnp.float32),
  )(part4, mask3, w1, w2, b2)
  return out.reshape(B * S, D)


@jax.jit
def kernel(vision_trace_feat, vision_trace_mask, W, b):
  zeros = jnp.zeros((ROWS_PER_B, D), jnp.float32)
  partials = _sc_kernel(vision_trace_feat, vision_trace_mask.astype(jnp.int32),
                        zeros)
  return _tc_finish(partials, vision_trace_mask, W, b)
